# baseline (speedup 1.0000x reference)
import functools

import jax
import jax.numpy as jnp
from jax import lax
from jax.experimental import pallas as pl
from jax.experimental.pallas import tpu as pltpu
from jax.experimental.pallas import tpu_sc as plsc

BATCH = 16384
INPUT_BITS = 1024
POS_BITS = 16
N_TAPS = 8
RAM_SIZE = 256
LANES = 16

TC_ROWS = 9216
SC_ROWS = BATCH - TC_ROWS
TC_BLK = 1024

NUM_WORKERS = 32
ROWS_PER_WORKER = SC_ROWS // NUM_WORKERS
CHUNK_ROWS = 32
N_CHUNKS = ROWS_PER_WORKER // CHUNK_ROWS
GROUPS_PER_CHUNK = CHUNK_ROWS // LANES


def _sc_body(
    query_hbm,
    connf_hbm,
    ram_hbm,
    out_hbm,
    qbuf0,
    qbuf1,
    connv,
    ramv,
    outv,
    sem0,
    sem1,
    sems,
):
    wid = lax.axis_index("s") * 2 + lax.axis_index("c")
    base = wid * ROWS_PER_WORKER

    def start(i, buf, sem):
        row0 = TC_ROWS + base + i * CHUNK_ROWS
        pltpu.async_copy(query_hbm.at[pl.ds(row0, CHUNK_ROWS), :], buf, sem)

    def drain(buf, sem):
        pltpu.make_async_copy(
            query_hbm.at[pl.ds(0, CHUNK_ROWS), :], buf, sem
        ).wait()

    start(0, qbuf0, sem0)
    pltpu.async_copy(connf_hbm, connv, sems)
    pltpu.async_copy(ram_hbm, ramv, sems)
    pltpu.make_async_copy(connf_hbm, connv, sems).wait()
    pltpu.make_async_copy(ram_hbm, ramv, sems).wait()

    def compute(i, buf):
        rows = [lax.iota(jnp.int32, 16) + (g * LANES) for g in range(GROUPS_PER_CHUNK)]
        pos = [jnp.zeros((16,), jnp.float32) for _ in range(GROUPS_PER_CHUNK)]
        for n in range(POS_BITS):
            addr = [jnp.zeros((16,), jnp.int32) for _ in range(GROUPS_PER_CHUNK)]
            for k in range(N_TAPS):
                c_vec = connv[pl.ds((n * N_TAPS + k) * LANES, LANES)]
                for g in range(GROUPS_PER_CHUNK):
                    bits = plsc.load_gather(buf, [rows[g], c_vec])
                    addr[g] = addr[g] + addr[g] + bits
            for g in range(GROUPS_PER_CHUNK):
                enc = plsc.load_gather(ramv, [addr[g] + (n * RAM_SIZE)])
                pos[g] = pos[g] + enc * float(2 ** (POS_BITS - 1 - n))
        for g in range(GROUPS_PER_CHUNK):
            outv[pl.ds(i * CHUNK_ROWS + g * LANES, LANES)] = jnp.minimum(
                pos[g], 32767.0
            )

    @pl.loop(0, N_CHUNKS - (N_CHUNKS % 2), step=2)
    def _chunk(i):
        start(i + 1, qbuf1, sem1)
        drain(qbuf0, sem0)
        compute(i, qbuf0)

        @pl.when(i + 2 < N_CHUNKS)
        def _():
            start(i + 2, qbuf0, sem0)

        drain(qbuf1, sem1)
        compute(i + 1, qbuf1)

    if N_CHUNKS % 2:
        drain(qbuf0, sem0)
        compute(N_CHUNKS - 1, qbuf0)

    pltpu.sync_copy(outv, out_hbm.at[pl.ds(base, ROWS_PER_WORKER)])


@functools.cache
def _sc_call():
    return functools.partial(
        pl.kernel,
        out_type=jax.ShapeDtypeStruct((SC_ROWS,), jnp.float32),
        mesh=plsc.VectorSubcoreMesh(
            core_axis_name="c", subcore_axis_name="s", num_cores=2, num_subcores=16
        ),
        compiler_params=pltpu.CompilerParams(
            needs_layout_passes=False, use_tc_tiling_on_sc=True
        ),
        scratch_types=[
            pltpu.VMEM((CHUNK_ROWS, INPUT_BITS), jnp.int32),
            pltpu.VMEM((CHUNK_ROWS, INPUT_BITS), jnp.int32),
            pltpu.VMEM((POS_BITS * N_TAPS * LANES,), jnp.int32),
            pltpu.VMEM((POS_BITS * RAM_SIZE,), jnp.float32),
            pltpu.VMEM((ROWS_PER_WORKER,), jnp.float32),
            pltpu.SemaphoreType.DMA,
            pltpu.SemaphoreType.DMA,
            pltpu.SemaphoreType.DMA,
        ],
    )(_sc_body)


def _tc_body(q_ref, conn_ref, ram_ref, out_ref):
    q = q_ref[...]
    conn = conn_ref[...]
    ram = ram_ref[...]
    col_io = lax.broadcasted_iota(jnp.int32, (POS_BITS, INPUT_BITS), 1)
    w = jnp.zeros((POS_BITS, INPUT_BITS), jnp.float32)
    for k in range(N_TAPS):
        ck = conn[k, :][:, None]
        w = w + jnp.where(col_io == ck, float(2 ** (N_TAPS - 1 - k)), 0.0)
    addr = lax.dot_general(
        w.astype(jnp.bfloat16),
        q.astype(jnp.bfloat16),
        (((1,), (1,)), ((), ())),
        preferred_element_type=jnp.float32,
    ).astype(jnp.int32)
    a_io = lax.broadcasted_iota(jnp.int32, (RAM_SIZE, 16), 0)
    w_io = lax.broadcasted_iota(jnp.int32, (RAM_SIZE, 16), 1)
    pmat = jnp.where((a_io >> 4) == w_io, (1 << (a_io & 15)).astype(jnp.float32), 0.0)
    words = jnp.dot(
        ram.astype(jnp.bfloat16),
        pmat.astype(jnp.bfloat16),
        preferred_element_type=jnp.float32,
    ).astype(jnp.int32)
    hi = addr >> 4
    lo = addr & 15
    word = jnp.zeros_like(addr)
    for h in range(16):
        word = word + jnp.where(hi == h, words[:, h][:, None], 0)
    bit = (word >> lo) & 1
    n_io = lax.broadcasted_iota(jnp.int32, bit.shape, 0)
    pos = jnp.sum(bit << (15 - n_io), axis=0)
    out_ref[...] = jnp.minimum(pos.astype(jnp.float32), 32767.0)


def _tc_call(query, conn_t, ram):
    return pl.pallas_call(
        _tc_body,
        grid=(TC_ROWS // TC_BLK,),
        in_specs=[
            pl.BlockSpec((TC_BLK, INPUT_BITS), lambda i: (i, 0)),
            pl.BlockSpec((N_TAPS, POS_BITS), lambda i: (0, 0)),
            pl.BlockSpec((POS_BITS, RAM_SIZE), lambda i: (0, 0)),
        ],
        out_specs=pl.BlockSpec((TC_BLK,), lambda i: (i,)),
        out_shape=jax.ShapeDtypeStruct((TC_ROWS,), jnp.float32),
    )(query, conn_t, ram)


def kernel(query, connections, ram_memory):
    conn_b = jnp.broadcast_to(
        connections.reshape(POS_BITS * N_TAPS, 1), (POS_BITS * N_TAPS, LANES)
    ).reshape(-1)
    sc_out = _sc_call()(query, conn_b, ram_memory.reshape(-1))
    tc_out = _tc_call(query, connections.T, ram_memory)
    return jnp.concatenate([tc_out, sc_out])


# speedup vs baseline: 1.0537x; 1.0537x over previous
import functools

import jax
import jax.numpy as jnp
from jax import lax
from jax.experimental import pallas as pl
from jax.experimental.pallas import tpu as pltpu
from jax.experimental.pallas import tpu_sc as plsc

BATCH = 16384
INPUT_BITS = 1024
POS_BITS = 16
N_TAPS = 8
RAM_SIZE = 256
LANES = 16

TC_ROWS = 10240
SC_ROWS = BATCH - TC_ROWS
TC_BLK = 1024

NUM_WORKERS = 32
ROWS_PER_WORKER = SC_ROWS // NUM_WORKERS
CHUNK_ROWS = 32
N_CHUNKS = ROWS_PER_WORKER // CHUNK_ROWS
GROUPS_PER_CHUNK = CHUNK_ROWS // LANES


def _sc_body(
    query_hbm,
    connf_hbm,
    ram_hbm,
    out_hbm,
    qbuf0,
    qbuf1,
    connv,
    ramv,
    outv,
    sem0,
    sem1,
    sems,
):
    wid = lax.axis_index("s") * 2 + lax.axis_index("c")
    base = wid * ROWS_PER_WORKER

    def start(i, buf, sem):
        row0 = TC_ROWS + base + i * CHUNK_ROWS
        pltpu.async_copy(query_hbm.at[pl.ds(row0, CHUNK_ROWS), :], buf, sem)

    def drain(buf, sem):
        pltpu.make_async_copy(
            query_hbm.at[pl.ds(0, CHUNK_ROWS), :], buf, sem
        ).wait()

    start(0, qbuf0, sem0)
    pltpu.async_copy(connf_hbm, connv, sems)
    pltpu.async_copy(ram_hbm, ramv, sems)
    pltpu.make_async_copy(connf_hbm, connv, sems).wait()
    pltpu.make_async_copy(ram_hbm, ramv, sems).wait()

    def compute(i, buf):
        rows = [lax.iota(jnp.int32, 16) + (g * LANES) for g in range(GROUPS_PER_CHUNK)]
        pos = [jnp.zeros((16,), jnp.float32) for _ in range(GROUPS_PER_CHUNK)]
        for n in range(POS_BITS):
            addr = [jnp.zeros((16,), jnp.int32) for _ in range(GROUPS_PER_CHUNK)]
            for k in range(N_TAPS):
                c_vec = connv[pl.ds((n * N_TAPS + k) * LANES, LANES)]
                for g in range(GROUPS_PER_CHUNK):
                    bits = plsc.load_gather(buf, [rows[g], c_vec])
                    addr[g] = addr[g] + addr[g] + bits
            for g in range(GROUPS_PER_CHUNK):
                enc = plsc.load_gather(ramv, [addr[g] + (n * RAM_SIZE)])
                pos[g] = pos[g] + enc * float(2 ** (POS_BITS - 1 - n))
        for g in range(GROUPS_PER_CHUNK):
            outv[pl.ds(i * CHUNK_ROWS + g * LANES, LANES)] = jnp.minimum(
                pos[g], 32767.0
            )

    @pl.loop(0, N_CHUNKS - (N_CHUNKS % 2), step=2)
    def _chunk(i):
        start(i + 1, qbuf1, sem1)
        drain(qbuf0, sem0)
        compute(i, qbuf0)

        @pl.when(i + 2 < N_CHUNKS)
        def _():
            start(i + 2, qbuf0, sem0)

        drain(qbuf1, sem1)
        compute(i + 1, qbuf1)

    if N_CHUNKS % 2:
        drain(qbuf0, sem0)
        compute(N_CHUNKS - 1, qbuf0)

    pltpu.sync_copy(outv, out_hbm.at[pl.ds(base, ROWS_PER_WORKER)])


@functools.cache
def _sc_call():
    return functools.partial(
        pl.kernel,
        out_type=jax.ShapeDtypeStruct((SC_ROWS,), jnp.float32),
        mesh=plsc.VectorSubcoreMesh(
            core_axis_name="c", subcore_axis_name="s", num_cores=2, num_subcores=16
        ),
        compiler_params=pltpu.CompilerParams(
            needs_layout_passes=False, use_tc_tiling_on_sc=True
        ),
        scratch_types=[
            pltpu.VMEM((CHUNK_ROWS, INPUT_BITS), jnp.int32),
            pltpu.VMEM((CHUNK_ROWS, INPUT_BITS), jnp.int32),
            pltpu.VMEM((POS_BITS * N_TAPS * LANES,), jnp.int32),
            pltpu.VMEM((POS_BITS * RAM_SIZE,), jnp.float32),
            pltpu.VMEM((ROWS_PER_WORKER,), jnp.float32),
            pltpu.SemaphoreType.DMA,
            pltpu.SemaphoreType.DMA,
            pltpu.SemaphoreType.DMA,
        ],
    )(_sc_body)


def _tc_body(q_ref, conn_ref, ram_ref, out_ref):
    q = q_ref[...]
    conn = conn_ref[...]
    ram = ram_ref[...]
    col_io = lax.broadcasted_iota(jnp.int32, (POS_BITS, INPUT_BITS), 1)
    w = jnp.zeros((POS_BITS, INPUT_BITS), jnp.float32)
    for k in range(N_TAPS):
        ck = conn[k, :][:, None]
        w = w + jnp.where(col_io == ck, float(2 ** (N_TAPS - 1 - k)), 0.0)
    addr = lax.dot_general(
        w.astype(jnp.bfloat16),
        q.astype(jnp.bfloat16),
        (((1,), (1,)), ((), ())),
        preferred_element_type=jnp.float32,
    ).astype(jnp.int32)
    a_io = lax.broadcasted_iota(jnp.int32, (RAM_SIZE, 16), 0)
    w_io = lax.broadcasted_iota(jnp.int32, (RAM_SIZE, 16), 1)
    pmat = jnp.where((a_io >> 4) == w_io, (1 << (a_io & 15)).astype(jnp.float32), 0.0)
    words = jnp.dot(
        ram.astype(jnp.bfloat16),
        pmat.astype(jnp.bfloat16),
        preferred_element_type=jnp.float32,
    ).astype(jnp.int32)
    hi = addr >> 4
    lo = addr & 15
    word = jnp.zeros_like(addr)
    for h in range(16):
        word = word + jnp.where(hi == h, words[:, h][:, None], 0)
    bit = (word >> lo) & 1
    n_io = lax.broadcasted_iota(jnp.int32, bit.shape, 0)
    pos = jnp.sum(bit << (15 - n_io), axis=0)
    out_ref[...] = jnp.minimum(pos.astype(jnp.float32), 32767.0)


def _tc_call(query, conn_t, ram):
    return pl.pallas_call(
        _tc_body,
        grid=(TC_ROWS // TC_BLK,),
        in_specs=[
            pl.BlockSpec((TC_BLK, INPUT_BITS), lambda i: (i, 0)),
            pl.BlockSpec((N_TAPS, POS_BITS), lambda i: (0, 0)),
            pl.BlockSpec((POS_BITS, RAM_SIZE), lambda i: (0, 0)),
        ],
        out_specs=pl.BlockSpec((TC_BLK,), lambda i: (i,)),
        out_shape=jax.ShapeDtypeStruct((TC_ROWS,), jnp.float32),
    )(query, conn_t, ram)


def kernel(query, connections, ram_memory):
    conn_b = jnp.broadcast_to(
        connections.reshape(POS_BITS * N_TAPS, 1), (POS_BITS * N_TAPS, LANES)
    ).reshape(-1)
    sc_out = _sc_call()(query, conn_b, ram_memory.reshape(-1))
    tc_out = _tc_call(query, connections.T, ram_memory)
    return jnp.concatenate([tc_out, sc_out])
